# Initial kernel scaffold; baseline (speedup 1.0000x reference)
import jax
import jax.numpy as jnp
from jax.experimental import pallas as pl


def kernel(atom_pos, dist_exp, gaussians, atom_emb_table, W_bil, b_bil, Wc0, bc0, Wc1, bc1, Wc2, bc2, Wg0, bg0, Wg1, bg1, Wg2, bg2, Wm0, bm0, Wm1, bm1, Ws0, bs0, Ws1, bs1, Wp, bp, atom_types, edge_index):
    raise NotImplementedError("write your pallas kernel here")


# speedup vs baseline: 4.9487x; 4.9487x over previous
import functools

import jax
import jax.numpy as jnp
from jax import lax
from jax.experimental import pallas as pl
from jax.experimental.pallas import tpu as pltpu
from jax.experimental.pallas import tpu_sc as plsc

N = 10000
E = 320000
NACC = 10240
BLK = 128
NTILES = 32
WIN = 128
NWIN = 80
CH = 8
NCHUNK = NWIN // CH
EPT = NWIN * WIN
EPAD = NTILES * EPT


def _deg_body(src_hbm, dst_hbm, degs_hbm, degd_hbm,
              dsrc, ddst, srcv, dstv, zdeg, ones1):
    c = lax.axis_index("c")
    s = lax.axis_index("s")
    wid = c * 16 + s
    z16 = jnp.zeros((16,), jnp.float32)
    o16 = jnp.full((16,), 1.0, jnp.float32)
    rpt = NACC // 16
    for i in range(rpt // 16):
        zdeg[pl.ds(i * 16, 16)] = z16
    for i in range(WIN // 16):
        ones1[pl.ds(i * 16, 16)] = o16
    pltpu.sync_copy(zdeg, dsrc.at[pl.ds(s * rpt, rpt)])
    pltpu.sync_copy(zdeg, ddst.at[pl.ds(s * rpt, rpt)])
    plsc.subcore_barrier()

    def chunk(ch, carry):
        pltpu.sync_copy(src_hbm.at[wid, pl.ds(ch * CH, CH)], srcv)
        pltpu.sync_copy(dst_hbm.at[wid, pl.ds(ch * CH, CH)], dstv)

        def win(w, c2):
            pltpu.sync_copy(ones1, dsrc.at[srcv.at[w]], add=True)
            pltpu.sync_copy(ones1, ddst.at[dstv.at[w]], add=True)
            return c2

        return lax.fori_loop(0, CH, win, carry)

    lax.fori_loop(0, NCHUNK, chunk, 0)
    plsc.subcore_barrier()
    pltpu.sync_copy(dsrc.at[pl.ds(s * rpt, rpt)],
                    degs_hbm.at[pl.ds(c * NACC + s * rpt, rpt)])
    pltpu.sync_copy(ddst.at[pl.ds(s * rpt, rpt)],
                    degd_hbm.at[pl.ds(c * NACC + s * rpt, rpt)])


def _segsum_body(feat_hbm, src_hbm, dst_hbm, out_hbm,
                 acc, srcv, dstv, rows, zbuf, sem):
    width = rows.shape[1]
    c = lax.axis_index("c")
    s = lax.axis_index("s")
    wid = c * 16 + s
    z16 = jnp.zeros((16,), jnp.float32)
    for i in range(16):
        for j in range(width // 16):
            zbuf[i, pl.ds(j * 16, 16)] = z16
    base = s * (NACC // 16)
    for k in range(NACC // 16 // 16):
        pltpu.sync_copy(zbuf, acc.at[pl.ds(base + k * 16, 16)])
    plsc.subcore_barrier()

    def chunk(ch, carry):
        pltpu.sync_copy(src_hbm.at[wid, pl.ds(ch * CH, CH)], srcv)
        pltpu.sync_copy(dst_hbm.at[wid, pl.ds(ch * CH, CH)], dstv)

        def win(w, c2):
            pltpu.async_copy(feat_hbm.at[srcv.at[w]], rows, sem).wait()
            pltpu.sync_copy(rows, acc.at[dstv.at[w]], add=True)
            return c2

        return lax.fori_loop(0, CH, win, carry)

    lax.fori_loop(0, NCHUNK, chunk, 0)
    plsc.subcore_barrier()
    rpt = NACC // 16
    pltpu.sync_copy(acc.at[pl.ds(s * rpt, rpt)],
                    out_hbm.at[c, pl.ds(s * rpt, rpt)])


def _make_mesh():
    return plsc.VectorSubcoreMesh(core_axis_name="c", subcore_axis_name="s",
                                  num_cores=2, num_subcores=16)


def _degrees(src3, dst3):
    return pl.kernel(
        _deg_body,
        out_type=[
            jax.ShapeDtypeStruct((2 * NACC,), jnp.float32),
            jax.ShapeDtypeStruct((2 * NACC,), jnp.float32),
        ],
        mesh=_make_mesh(),
        scratch_types=[
            pltpu.VMEM_SHARED((NACC,), jnp.float32),
            pltpu.VMEM_SHARED((NACC,), jnp.float32),
            pltpu.VMEM((CH, WIN), jnp.int32),
            pltpu.VMEM((CH, WIN), jnp.int32),
            pltpu.VMEM((NACC // 16,), jnp.float32),
            pltpu.VMEM((WIN,), jnp.float32),
        ],
        name="degrees",
    )(src3, dst3)


def _segsum(feat, src3, dst3):
    width = feat.shape[1]
    return pl.kernel(
        _segsum_body,
        out_type=jax.ShapeDtypeStruct((2, NACC, width), jnp.float32),
        mesh=_make_mesh(),
        scratch_types=[
            pltpu.VMEM_SHARED((NACC, width), jnp.float32),
            pltpu.VMEM((CH, WIN), jnp.int32),
            pltpu.VMEM((CH, WIN), jnp.int32),
            pltpu.VMEM((WIN, width), jnp.float32),
            pltpu.VMEM((16, width), jnp.float32),
            pltpu.SemaphoreType.DMA,
        ],
        name="segsum",
    )(feat, src3, dst3)


def _softplus(x):
    return jnp.maximum(x, 0.0) + jnp.log1p(jnp.exp(-jnp.abs(x)))


def _lrelu(x):
    return jnp.where(x >= 0, x, 0.01 * x)


def _emb_bilinear_kernel(types_ref, z_ref, table_ref, wflat_ref,
                         emb_ref, feat_ref):
    t = types_ref[...]
    iota = lax.broadcasted_iota(jnp.int32, (BLK, 128), 1)
    oh = (t == iota).astype(jnp.float32)
    emb = jnp.dot(oh, table_ref[...],
                  preferred_element_type=jnp.float32)
    z = z_ref[...]
    p = (z[:, :, None] * emb[:, None, :]).reshape(BLK, 64 * 128)
    feat = jnp.dot(p, wflat_ref[...],
                   preferred_element_type=jnp.float32)
    emb_ref[...] = emb
    feat_ref[...] = feat


def _post1_kernel(agg_ref, degp_ref, fsrc_ref, pos_ref, emb_ref,
                  bbil_ref, wc0_ref, bc0_ref, wc1_ref, bc1_ref,
                  wc2_ref, bc2_ref, wg0p_ref, wg0e_ref,
                  pred_ref, y0_ref, nsd_ref):
    agg = agg_ref[0] + agg_ref[1]
    x = _softplus(agg - fsrc_ref[...]) + bbil_ref[...]
    h = _softplus(jnp.dot(x, wc0_ref[...], preferred_element_type=jnp.float32)
                  + bc0_ref[...])
    h = _softplus(jnp.dot(h, wc1_ref[...], preferred_element_type=jnp.float32)
                  + bc1_ref[...])
    h = _softplus(jnp.dot(h, wc2_ref[...], preferred_element_type=jnp.float32)
                  + bc2_ref[...])
    pred_ref[...] = h
    degp = degp_ref[...]
    ns = lax.rsqrt(jnp.maximum(degp[:, 0:1] + degp[:, 1:2], 1.0))
    nd = lax.rsqrt(jnp.maximum(degp[:, 2:3] + degp[:, 3:4], 1.0))
    y0 = jnp.dot(pos_ref[...], wg0p_ref[...],
                 preferred_element_type=jnp.float32)
    y0 = y0 + jnp.dot(emb_ref[...], wg0e_ref[...],
                      preferred_element_type=jnp.float32)
    y0_ref[...] = y0 * ns
    nsd_ref[...] = jnp.concatenate(
        [jnp.broadcast_to(ns, (BLK, 8)), jnp.broadcast_to(nd, (BLK, 8))],
        axis=1)


def _gconv_mid_kernel(agg_ref, nsd_ref, bg_ref, wnext_ref, out_ref):
    agg = agg_ref[0] + agg_ref[1]
    nd = nsd_ref[...][:, 8:9]
    h = _softplus(agg * nd + bg_ref[...])
    out_ref[...] = jnp.dot(h, wnext_ref[...],
                           preferred_element_type=jnp.float32) \
        * nsd_ref[...][:, 0:1]


def _gconv_last_kernel(agg_ref, nsd_ref, bg_ref, out_ref):
    agg = agg_ref[0] + agg_ref[1]
    nd = nsd_ref[...][:, 8:9]
    out_ref[...] = _softplus(agg * nd + bg_ref[...])


def _vae_kernel(feat_ref, gauss_ref, wm0_ref, bm0_ref, wm1_ref, bm1_ref,
                ws0_ref, bs0_ref, ws1_ref, bs1_ref, wp_ref, bp_ref,
                vae_ref, pos_ref):
    f = feat_ref[...]
    mean = _lrelu(jnp.dot(f, wm0_ref[...], preferred_element_type=jnp.float32)
                  + bm0_ref[...])
    mean = _lrelu(jnp.dot(mean, wm1_ref[...],
                          preferred_element_type=jnp.float32) + bm1_ref[...])
    std = _lrelu(jnp.dot(f, ws0_ref[...], preferred_element_type=jnp.float32)
                 + bs0_ref[...])
    std = _lrelu(jnp.dot(std, ws1_ref[...],
                         preferred_element_type=jnp.float32) + bs1_ref[...])
    vae = gauss_ref[...] * std + mean
    vae_ref[...] = vae
    pos_ref[...] = jnp.dot(vae, wp_ref[...],
                           preferred_element_type=jnp.float32) + bp_ref[...]


def _row_spec(width):
    return pl.BlockSpec((BLK, width), lambda i: (i, 0))


def _full_spec(shape):
    nd = len(shape)
    return pl.BlockSpec(shape, lambda i, _n=nd: (0,) * _n)


def _part_spec(width):
    return pl.BlockSpec((2, BLK, width), lambda i: (0, i, 0))


GRID = NACC // BLK


def _call(body, in_specs, out_specs, out_shapes, *args):
    return pl.pallas_call(
        body, grid=(GRID,), in_specs=in_specs, out_specs=out_specs,
        out_shape=out_shapes)(*args)


def kernel(atom_pos, dist_exp, gaussians, atom_emb_table, W_bil, b_bil,
           Wc0, bc0, Wc1, bc1, Wc2, bc2,
           Wg0, bg0, Wg1, bg1, Wg2, bg2,
           Wm0, bm0, Wm1, bm1, Ws0, bs0, Ws1, bs1,
           Wp, bp, atom_types, edge_index):
    f32 = jnp.float32
    padn = NACC - N
    pos8 = jnp.pad(atom_pos.astype(f32), ((0, padn), (0, 5)))
    z64 = jnp.pad(dist_exp.astype(f32), ((0, padn), (0, 0)))
    gauss = jnp.pad(gaussians.astype(f32), ((0, padn), (0, 0)))
    types = jnp.pad(atom_types.astype(jnp.int32), (0, padn)).reshape(NACC, 1)
    table = jnp.pad(atom_emb_table.astype(f32), ((0, 28), (0, 0)))
    wflat = W_bil.astype(f32).reshape(64 * 128, 128)
    wg0p = jnp.pad(Wg0.astype(f32)[0:3], ((0, 5), (0, 0)))
    wg0e = Wg0.astype(f32)[3:131]
    wpp = jnp.pad(Wp.astype(f32), ((0, 0), (0, 125)))
    bpp = jnp.pad(bp.astype(f32), (0, 125)).reshape(1, 128)
    b2 = lambda b: b.astype(f32).reshape(1, 128)

    src = edge_index[0].astype(jnp.int32)
    dst = edge_index[1].astype(jnp.int32)
    fill = (N + 48 + (jnp.arange(EPAD - E, dtype=jnp.int32) % 64))
    src3 = jnp.concatenate([src, fill]).reshape(NTILES, NWIN, WIN)
    dst3 = jnp.concatenate([dst, fill]).reshape(NTILES, NWIN, WIN)

    emb, fsrc = _call(
        _emb_bilinear_kernel,
        [_row_spec(1), _row_spec(64), _full_spec((128, 128)),
         _full_spec((64 * 128, 128))],
        [_row_spec(128), _row_spec(128)],
        [jax.ShapeDtypeStruct((NACC, 128), f32),
         jax.ShapeDtypeStruct((NACC, 128), f32)],
        types, z64, table, wflat)

    degs_raw, degd_raw = _degrees(src3, dst3)
    agg0p = _segsum(fsrc, src3, dst3)
    ds2 = degs_raw.reshape(2, NACC)
    dd2 = degd_raw.reshape(2, NACC)
    degp = jnp.stack(
        [ds2[0], ds2[1], dd2[0], dd2[1],
         ds2[0], ds2[1], dd2[0], dd2[1]], axis=-1)

    pred, y, nsd = _call(
        _post1_kernel,
        [_part_spec(128), _row_spec(8), _row_spec(128),
         _row_spec(8), _row_spec(128), _full_spec((1, 128)),
         _full_spec((128, 128)), _full_spec((1, 128)),
         _full_spec((128, 128)), _full_spec((1, 128)),
         _full_spec((128, 128)), _full_spec((1, 128)),
         _full_spec((8, 128)), _full_spec((128, 128))],
        [_row_spec(128), _row_spec(128), _row_spec(16)],
        [jax.ShapeDtypeStruct((NACC, 128), f32),
         jax.ShapeDtypeStruct((NACC, 128), f32),
         jax.ShapeDtypeStruct((NACC, 16), f32)],
        agg0p, degp, fsrc, pos8, emb, b2(b_bil),
        Wc0.astype(f32), b2(bc0), Wc1.astype(f32), b2(bc1),
        Wc2.astype(f32), b2(bc2), wg0p, wg0e)

    for bg, wnext in ((bg0, Wg1), (bg1, Wg2)):
        aggy = _segsum(y, src3, dst3)
        y = _call(
            _gconv_mid_kernel,
            [_part_spec(128), _row_spec(16), _full_spec((1, 128)),
             _full_spec((128, 128))],
            _row_spec(128),
            jax.ShapeDtypeStruct((NACC, 128), f32),
            aggy, nsd, b2(bg), wnext.astype(f32))
    aggy = _segsum(y, src3, dst3)
    feat = _call(
        _gconv_last_kernel,
        [_part_spec(128), _row_spec(16), _full_spec((1, 128))],
        _row_spec(128),
        jax.ShapeDtypeStruct((NACC, 128), f32),
        aggy, nsd, b2(bg2))

    vae, pospred = _call(
        _vae_kernel,
        [_row_spec(128), _row_spec(128),
         _full_spec((128, 128)), _full_spec((1, 128)),
         _full_spec((128, 128)), _full_spec((1, 128)),
         _full_spec((128, 128)), _full_spec((1, 128)),
         _full_spec((128, 128)), _full_spec((1, 128)),
         _full_spec((128, 128)), _full_spec((1, 128))],
        [_row_spec(128), _row_spec(128)],
        [jax.ShapeDtypeStruct((NACC, 128), f32),
         jax.ShapeDtypeStruct((NACC, 128), f32)],
        feat, gauss, Wm0.astype(f32), b2(bm0), Wm1.astype(f32), b2(bm1),
        Ws0.astype(f32), b2(bs0), Ws1.astype(f32), b2(bs1), wpp, bpp)

    return (pred[:N], pospred[:N, :3], vae[:N])


# speedup vs baseline: 6.0105x; 1.2146x over previous
import functools

import jax
import jax.numpy as jnp
from jax import lax
from jax.experimental import pallas as pl
from jax.experimental.pallas import tpu as pltpu
from jax.experimental.pallas import tpu_sc as plsc

N = 10000
E = 320000
NACC = 10240
BLK = 128
NTILES = 32
WIN = 128
NWIN = 80
CH = 8
NCHUNK = NWIN // CH
EPT = NWIN * WIN
EPAD = NTILES * EPT


def _deg_body(src_hbm, dst_hbm, degs_hbm, degd_hbm,
              dsrc, ddst, srcv, dstv, zdeg, ones1):
    c = lax.axis_index("c")
    s = lax.axis_index("s")
    wid = c * 16 + s
    z16 = jnp.zeros((16,), jnp.float32)
    o16 = jnp.full((16,), 1.0, jnp.float32)
    rpt = NACC // 16
    for i in range(rpt // 16):
        zdeg[pl.ds(i * 16, 16)] = z16
    for i in range(WIN // 16):
        ones1[pl.ds(i * 16, 16)] = o16
    pltpu.sync_copy(zdeg, dsrc.at[pl.ds(s * rpt, rpt)])
    pltpu.sync_copy(zdeg, ddst.at[pl.ds(s * rpt, rpt)])
    plsc.subcore_barrier()

    def chunk(ch, carry):
        pltpu.sync_copy(src_hbm.at[wid, pl.ds(ch * CH, CH)], srcv)
        pltpu.sync_copy(dst_hbm.at[wid, pl.ds(ch * CH, CH)], dstv)

        def win(w, c2):
            pltpu.sync_copy(ones1, dsrc.at[srcv.at[w]], add=True)
            pltpu.sync_copy(ones1, ddst.at[dstv.at[w]], add=True)
            return c2

        return lax.fori_loop(0, CH, win, carry)

    lax.fori_loop(0, NCHUNK, chunk, 0)
    plsc.subcore_barrier()
    pltpu.sync_copy(dsrc.at[pl.ds(s * rpt, rpt)],
                    degs_hbm.at[pl.ds(c * NACC + s * rpt, rpt)])
    pltpu.sync_copy(ddst.at[pl.ds(s * rpt, rpt)],
                    degd_hbm.at[pl.ds(c * NACC + s * rpt, rpt)])


def _segsum_body(feat_hbm, src_hbm, dst_hbm, out_hbm,
                 acc, srcv, dstv, rows0, rows1, zbuf, sem0, sem1):
    width = rows0.shape[1]
    c = lax.axis_index("c")
    s = lax.axis_index("s")
    wid = c * 16 + s
    z16 = jnp.zeros((16,), jnp.float32)
    for i in range(8):
        for j in range(width // 16):
            zbuf[i, pl.ds(j * 16, 16)] = z16
    base = s * (NACC // 16)
    for k in range(NACC // 16 // 8):
        pltpu.sync_copy(zbuf, acc.at[pl.ds(base + k * 8, 8)])
    plsc.subcore_barrier()
    bufs = (rows0, rows1)
    sems = (sem0, sem1)

    def chunk(ch, carry):
        pltpu.sync_copy(src_hbm.at[wid, pl.ds(ch * CH, CH)], srcv)
        pltpu.sync_copy(dst_hbm.at[wid, pl.ds(ch * CH, CH)], dstv)
        cps = [pltpu.async_copy(feat_hbm.at[srcv.at[0]], rows0, sem0)]
        for w in range(CH):
            if w + 1 < CH:
                cps.append(pltpu.async_copy(
                    feat_hbm.at[srcv.at[w + 1]],
                    bufs[(w + 1) % 2], sems[(w + 1) % 2]))
            cps[w].wait()
            pltpu.sync_copy(bufs[w % 2], acc.at[dstv.at[w]], add=True)
        return carry

    lax.fori_loop(0, NCHUNK, chunk, 0)
    plsc.subcore_barrier()
    rpt = NACC // 16
    pltpu.sync_copy(acc.at[pl.ds(s * rpt, rpt)],
                    out_hbm.at[c, pl.ds(s * rpt, rpt)])


def _make_mesh():
    return plsc.VectorSubcoreMesh(core_axis_name="c", subcore_axis_name="s",
                                  num_cores=2, num_subcores=16)


def _degrees(src3, dst3):
    return pl.kernel(
        _deg_body,
        out_type=[
            jax.ShapeDtypeStruct((2 * NACC,), jnp.float32),
            jax.ShapeDtypeStruct((2 * NACC,), jnp.float32),
        ],
        mesh=_make_mesh(),
        scratch_types=[
            pltpu.VMEM_SHARED((NACC,), jnp.float32),
            pltpu.VMEM_SHARED((NACC,), jnp.float32),
            pltpu.VMEM((CH, WIN), jnp.int32),
            pltpu.VMEM((CH, WIN), jnp.int32),
            pltpu.VMEM((NACC // 16,), jnp.float32),
            pltpu.VMEM((WIN,), jnp.float32),
        ],
        name="degrees",
    )(src3, dst3)


def _segsum(feat, src3, dst3):
    width = feat.shape[1]
    return pl.kernel(
        _segsum_body,
        out_type=jax.ShapeDtypeStruct((2, NACC, width), jnp.float32),
        mesh=_make_mesh(),
        scratch_types=[
            pltpu.VMEM_SHARED((NACC, width), jnp.float32),
            pltpu.VMEM((CH, WIN), jnp.int32),
            pltpu.VMEM((CH, WIN), jnp.int32),
            pltpu.VMEM((WIN, width), jnp.float32),
            pltpu.VMEM((WIN, width), jnp.float32),
            pltpu.VMEM((8, width), jnp.float32),
            pltpu.SemaphoreType.DMA,
            pltpu.SemaphoreType.DMA,
        ],
        name="segsum",
    )(feat, src3, dst3)


def _softplus(x):
    return jnp.maximum(x, 0.0) + jnp.log1p(jnp.exp(-jnp.abs(x)))


def _lrelu(x):
    return jnp.where(x >= 0, x, 0.01 * x)


def _emb_bilinear_kernel(types_ref, z_ref, table_ref, wflat_ref,
                         emb_ref, feat_ref):
    t = types_ref[...]
    iota = lax.broadcasted_iota(jnp.int32, (BLK, 128), 1)
    oh = (t == iota).astype(jnp.float32)
    emb = jnp.dot(oh, table_ref[...],
                  preferred_element_type=jnp.float32)
    z = z_ref[...]
    p = (z[:, :, None] * emb[:, None, :]).reshape(BLK, 64 * 128)
    feat = jnp.dot(p, wflat_ref[...],
                   preferred_element_type=jnp.float32)
    emb_ref[...] = emb
    feat_ref[...] = feat


def _post1_kernel(agg_ref, degp_ref, fsrc_ref, pos_ref, emb_ref,
                  bbil_ref, wc0_ref, bc0_ref, wc1_ref, bc1_ref,
                  wc2_ref, bc2_ref, wg0p_ref, wg0e_ref,
                  pred_ref, y0_ref, nsd_ref):
    agg = agg_ref[0] + agg_ref[1]
    x = _softplus(agg - fsrc_ref[...]) + bbil_ref[...]
    h = _softplus(jnp.dot(x, wc0_ref[...], preferred_element_type=jnp.float32)
                  + bc0_ref[...])
    h = _softplus(jnp.dot(h, wc1_ref[...], preferred_element_type=jnp.float32)
                  + bc1_ref[...])
    h = _softplus(jnp.dot(h, wc2_ref[...], preferred_element_type=jnp.float32)
                  + bc2_ref[...])
    pred_ref[...] = h
    degp = degp_ref[...]
    ns = lax.rsqrt(jnp.maximum(degp[:, 0:1] + degp[:, 1:2], 1.0))
    nd = lax.rsqrt(jnp.maximum(degp[:, 2:3] + degp[:, 3:4], 1.0))
    y0 = jnp.dot(pos_ref[...], wg0p_ref[...],
                 preferred_element_type=jnp.float32)
    y0 = y0 + jnp.dot(emb_ref[...], wg0e_ref[...],
                      preferred_element_type=jnp.float32)
    y0_ref[...] = y0 * ns
    nsd_ref[...] = jnp.concatenate(
        [jnp.broadcast_to(ns, (BLK, 8)), jnp.broadcast_to(nd, (BLK, 8))],
        axis=1)


def _gconv_mid_kernel(agg_ref, nsd_ref, bg_ref, wnext_ref, out_ref):
    agg = agg_ref[0] + agg_ref[1]
    nd = nsd_ref[...][:, 8:9]
    h = _softplus(agg * nd + bg_ref[...])
    out_ref[...] = jnp.dot(h, wnext_ref[...],
                           preferred_element_type=jnp.float32) \
        * nsd_ref[...][:, 0:1]


def _gconv_last_kernel(agg_ref, nsd_ref, bg_ref, out_ref):
    agg = agg_ref[0] + agg_ref[1]
    nd = nsd_ref[...][:, 8:9]
    out_ref[...] = _softplus(agg * nd + bg_ref[...])


def _vae_kernel(feat_ref, gauss_ref, wm0_ref, bm0_ref, wm1_ref, bm1_ref,
                ws0_ref, bs0_ref, ws1_ref, bs1_ref, wp_ref, bp_ref,
                vae_ref, pos_ref):
    f = feat_ref[...]
    mean = _lrelu(jnp.dot(f, wm0_ref[...], preferred_element_type=jnp.float32)
                  + bm0_ref[...])
    mean = _lrelu(jnp.dot(mean, wm1_ref[...],
                          preferred_element_type=jnp.float32) + bm1_ref[...])
    std = _lrelu(jnp.dot(f, ws0_ref[...], preferred_element_type=jnp.float32)
                 + bs0_ref[...])
    std = _lrelu(jnp.dot(std, ws1_ref[...],
                         preferred_element_type=jnp.float32) + bs1_ref[...])
    vae = gauss_ref[...] * std + mean
    vae_ref[...] = vae
    pos_ref[...] = jnp.dot(vae, wp_ref[...],
                           preferred_element_type=jnp.float32) + bp_ref[...]


def _row_spec(width):
    return pl.BlockSpec((BLK, width), lambda i: (i, 0))


def _full_spec(shape):
    nd = len(shape)
    return pl.BlockSpec(shape, lambda i, _n=nd: (0,) * _n)


def _part_spec(width):
    return pl.BlockSpec((2, BLK, width), lambda i: (0, i, 0))


GRID = NACC // BLK


def _call(body, in_specs, out_specs, out_shapes, *args):
    return pl.pallas_call(
        body, grid=(GRID,), in_specs=in_specs, out_specs=out_specs,
        out_shape=out_shapes)(*args)


def kernel(atom_pos, dist_exp, gaussians, atom_emb_table, W_bil, b_bil,
           Wc0, bc0, Wc1, bc1, Wc2, bc2,
           Wg0, bg0, Wg1, bg1, Wg2, bg2,
           Wm0, bm0, Wm1, bm1, Ws0, bs0, Ws1, bs1,
           Wp, bp, atom_types, edge_index):
    f32 = jnp.float32
    padn = NACC - N
    pos8 = jnp.pad(atom_pos.astype(f32), ((0, padn), (0, 5)))
    z64 = jnp.pad(dist_exp.astype(f32), ((0, padn), (0, 0)))
    gauss = jnp.pad(gaussians.astype(f32), ((0, padn), (0, 0)))
    types = jnp.pad(atom_types.astype(jnp.int32), (0, padn)).reshape(NACC, 1)
    table = jnp.pad(atom_emb_table.astype(f32), ((0, 28), (0, 0)))
    wflat = W_bil.astype(f32).reshape(64 * 128, 128)
    wg0p = jnp.pad(Wg0.astype(f32)[0:3], ((0, 5), (0, 0)))
    wg0e = Wg0.astype(f32)[3:131]
    wpp = jnp.pad(Wp.astype(f32), ((0, 0), (0, 125)))
    bpp = jnp.pad(bp.astype(f32), (0, 125)).reshape(1, 128)
    b2 = lambda b: b.astype(f32).reshape(1, 128)

    src = edge_index[0].astype(jnp.int32)
    dst = edge_index[1].astype(jnp.int32)
    fill = (N + 48 + (jnp.arange(EPAD - E, dtype=jnp.int32) % 64))
    src3 = jnp.concatenate([src, fill]).reshape(NTILES, NWIN, WIN)
    dst3 = jnp.concatenate([dst, fill]).reshape(NTILES, NWIN, WIN)

    emb, fsrc = _call(
        _emb_bilinear_kernel,
        [_row_spec(1), _row_spec(64), _full_spec((128, 128)),
         _full_spec((64 * 128, 128))],
        [_row_spec(128), _row_spec(128)],
        [jax.ShapeDtypeStruct((NACC, 128), f32),
         jax.ShapeDtypeStruct((NACC, 128), f32)],
        types, z64, table, wflat)

    degs_raw, degd_raw = _degrees(src3, dst3)
    agg0p = _segsum(fsrc, src3, dst3)
    ds2 = degs_raw.reshape(2, NACC)
    dd2 = degd_raw.reshape(2, NACC)
    degp = jnp.stack(
        [ds2[0], ds2[1], dd2[0], dd2[1],
         ds2[0], ds2[1], dd2[0], dd2[1]], axis=-1)

    pred, y, nsd = _call(
        _post1_kernel,
        [_part_spec(128), _row_spec(8), _row_spec(128),
         _row_spec(8), _row_spec(128), _full_spec((1, 128)),
         _full_spec((128, 128)), _full_spec((1, 128)),
         _full_spec((128, 128)), _full_spec((1, 128)),
         _full_spec((128, 128)), _full_spec((1, 128)),
         _full_spec((8, 128)), _full_spec((128, 128))],
        [_row_spec(128), _row_spec(128), _row_spec(16)],
        [jax.ShapeDtypeStruct((NACC, 128), f32),
         jax.ShapeDtypeStruct((NACC, 128), f32),
         jax.ShapeDtypeStruct((NACC, 16), f32)],
        agg0p, degp, fsrc, pos8, emb, b2(b_bil),
        Wc0.astype(f32), b2(bc0), Wc1.astype(f32), b2(bc1),
        Wc2.astype(f32), b2(bc2), wg0p, wg0e)

    for bg, wnext in ((bg0, Wg1), (bg1, Wg2)):
        aggy = _segsum(y, src3, dst3)
        y = _call(
            _gconv_mid_kernel,
            [_part_spec(128), _row_spec(16), _full_spec((1, 128)),
             _full_spec((128, 128))],
            _row_spec(128),
            jax.ShapeDtypeStruct((NACC, 128), f32),
            aggy, nsd, b2(bg), wnext.astype(f32))
    aggy = _segsum(y, src3, dst3)
    feat = _call(
        _gconv_last_kernel,
        [_part_spec(128), _row_spec(16), _full_spec((1, 128))],
        _row_spec(128),
        jax.ShapeDtypeStruct((NACC, 128), f32),
        aggy, nsd, b2(bg2))

    vae, pospred = _call(
        _vae_kernel,
        [_row_spec(128), _row_spec(128),
         _full_spec((128, 128)), _full_spec((1, 128)),
         _full_spec((128, 128)), _full_spec((1, 128)),
         _full_spec((128, 128)), _full_spec((1, 128)),
         _full_spec((128, 128)), _full_spec((1, 128)),
         _full_spec((128, 128)), _full_spec((1, 128))],
        [_row_spec(128), _row_spec(128)],
        [jax.ShapeDtypeStruct((NACC, 128), f32),
         jax.ShapeDtypeStruct((NACC, 128), f32)],
        feat, gauss, Wm0.astype(f32), b2(bm0), Wm1.astype(f32), b2(bm1),
        Ws0.astype(f32), b2(bs0), Ws1.astype(f32), b2(bs1), wpp, bpp)

    return (pred[:N], pospred[:N, :3], vae[:N])
